# baseline (speedup 1.0000x reference)
import functools

import jax
import jax.numpy as jnp
from jax import lax
from jax.experimental import pallas as pl
from jax.experimental.pallas import tpu as pltpu
from jax.experimental.pallas import tpu_sc as plsc

N = 10000
D = 256
DH = 128
N_PAD = 10240
EP = 163840
CH = 128
NCHUNK_CONV = 80
NCHUNK_HALF = 40
NCHUNK_DEG = 40
N_DUMMY = N_PAD - N
ROWS_PER_TILE = N_PAD // 16

_MESH = plsc.VectorSubcoreMesh(core_axis_name="c", subcore_axis_name="s")

_f32 = jnp.float32


def _deg_body(dst2d, zeros1d, ones_h, out0, out1, dst_idx, ones_v, counts):
    cid = lax.axis_index("c")
    sid = lax.axis_index("s")
    base = (cid * 16 + sid) * NCHUNK_DEG
    pltpu.sync_copy(dst2d.at[pl.ds(base, NCHUNK_DEG)], dst_idx)
    pltpu.sync_copy(ones_h, ones_v)
    pltpu.sync_copy(zeros1d, counts.at[pl.ds(sid * ROWS_PER_TILE, ROWS_PER_TILE)])
    plsc.subcore_barrier()

    def step(j, c):
        pltpu.sync_copy(ones_v, counts.at[dst_idx.at[j]], add=True)
        return c

    lax.fori_loop(0, NCHUNK_DEG, step, 0)
    plsc.subcore_barrier()

    @pl.when(cid == 0)
    def _():
        pltpu.sync_copy(counts.at[pl.ds(sid * ROWS_PER_TILE, ROWS_PER_TILE)],
                        out0.at[pl.ds(sid * ROWS_PER_TILE, ROWS_PER_TILE)])

    @pl.when(cid == 1)
    def _():
        pltpu.sync_copy(counts.at[pl.ds(sid * ROWS_PER_TILE, ROWS_PER_TILE)],
                        out1.at[pl.ds(sid * ROWS_PER_TILE, ROWS_PER_TILE)])


_deg_call = pl.kernel(
    _deg_body,
    out_type=(jax.ShapeDtypeStruct((N_PAD,), _f32),
              jax.ShapeDtypeStruct((N_PAD,), _f32)),
    mesh=_MESH,
    scratch_types=[
        pltpu.VMEM((NCHUNK_DEG, CH), jnp.int32),
        pltpu.VMEM((CH,), _f32),
        pltpu.VMEM_SHARED((N_PAD,), _f32),
    ],
)


def _conv_body(xs0, xs1, src2d, dst2d, zrows, out0, out1,
               src_idx, dst_idx, rb0, rb1, acc, sem0, sem1):
    cid = lax.axis_index("c")
    sid = lax.axis_index("s")
    row0 = sid * ROWS_PER_TILE
    pltpu.sync_copy(zrows, acc.at[pl.ds(row0, ROWS_PER_TILE)])
    plsc.subcore_barrier()

    def run(xs, out):
        for half in range(NCHUNK_CONV // NCHUNK_HALF):
            ibase = sid * NCHUNK_CONV + half * NCHUNK_HALF
            pltpu.sync_copy(src2d.at[pl.ds(ibase, NCHUNK_HALF)], src_idx)
            pltpu.sync_copy(dst2d.at[pl.ds(ibase, NCHUNK_HALF)], dst_idx)
            pltpu.async_copy(xs.at[src_idx.at[0]], rb0, sem0)

            def step(i, c):
                j0 = 2 * i
                pltpu.make_async_copy(xs.at[src_idx.at[j0]], rb0, sem0).wait()
                pltpu.async_copy(xs.at[src_idx.at[j0 + 1]], rb1, sem1)
                pltpu.sync_copy(rb0, acc.at[dst_idx.at[j0]], add=True)
                pltpu.make_async_copy(xs.at[src_idx.at[j0 + 1]], rb1, sem1).wait()

                @pl.when(i < NCHUNK_HALF // 2 - 1)
                def _():
                    pltpu.async_copy(xs.at[src_idx.at[j0 + 2]], rb0, sem0)

                pltpu.sync_copy(rb1, acc.at[dst_idx.at[j0 + 1]], add=True)
                return c

            lax.fori_loop(0, NCHUNK_HALF // 2, step, 0)
        plsc.subcore_barrier()
        pltpu.sync_copy(acc.at[pl.ds(row0, ROWS_PER_TILE)],
                        out.at[pl.ds(row0, ROWS_PER_TILE)])

    @pl.when(cid == 0)
    def _():
        run(xs0, out0)

    @pl.when(cid == 1)
    def _():
        run(xs1, out1)


_conv_call = pl.kernel(
    _conv_body,
    out_type=(jax.ShapeDtypeStruct((N_PAD, DH), _f32),
              jax.ShapeDtypeStruct((N_PAD, DH), _f32)),
    mesh=_MESH,
    scratch_types=[
        pltpu.VMEM((NCHUNK_HALF, CH), jnp.int32),
        pltpu.VMEM((NCHUNK_HALF, CH), jnp.int32),
        pltpu.VMEM((CH, DH), _f32),
        pltpu.VMEM((CH, DH), _f32),
        pltpu.VMEM_SHARED((N_PAD, DH), _f32),
        pltpu.SemaphoreType.DMA,
        pltpu.SemaphoreType.DMA,
    ],
)


BN = 1024
GRID = (N_PAD // BN,)
_HIGH = jax.lax.Precision.HIGHEST


def _rows(d):
    return pl.BlockSpec((BN, d), lambda i: (i, 0))


def _full(shape):
    return pl.BlockSpec(shape, lambda i: tuple(0 for _ in shape))


def _mm(a, b):
    return jnp.dot(a, b, preferred_element_type=_f32)


def _gelu(t):
    return t * 0.5 * (1.0 + lax.erf(t * 0.7071067811865476))


def _ln(t, g, b):
    m = jnp.mean(t, axis=1, keepdims=True)
    v = jnp.mean((t - m) ** 2, axis=1, keepdims=True)
    return (t - m) * lax.rsqrt(v + 1e-5) * g + b


def _tc_pre_a(x_ref, g1, bt1, Wm1, bm1, Wc1, Ws, bs, hv_o, zskip):
    xb = x_ref[...]
    h = _ln(xb, g1[...], bt1[...])
    h = _mm(h, Wm1[...]) + bm1[...]
    hv_o[...] = _mm(h, Wc1[...])
    zskip[...] = _mm(xb, Ws[...]) + bs[...]


def _tc_pre_b(hv, c0, c1, xs0, xs1, dinv_o):
    deg = c0[...] + c1[...] + 1.0
    dinv = lax.rsqrt(deg)
    dinv_o[...] = dinv
    xs0[...] = hv[...][:, :DH] * dinv
    xs1[...] = hv[...][:, DH:] * dinv


def _tc_mid(u0, u1, xs0, xs1, dinv, bc1, Wc2, xs2_0, xs2_1):
    dv = dinv[...]
    h0 = _gelu(dv * (u0[...] + xs0[...]) + bc1[...][:, :DH])
    h1 = _gelu(dv * (u1[...] + xs1[...]) + bc1[...][:, DH:])
    hv2 = _mm(h0, Wc2[...][:DH, :]) + _mm(h1, Wc2[...][DH:, :])
    xs2_0[...] = hv2[:, :DH] * dv
    xs2_1[...] = hv2[:, DH:] * dv


def _tc_post(u0, u1, xs0, xs1, dinv, bc2, g2, bt2, Wm2, bm2, zskip, z_ref):
    dv = dinv[...]
    y0 = dv * (u0[...] + xs0[...]) + bc2[...][:, :DH]
    y1 = dv * (u1[...] + xs1[...]) + bc2[...][:, DH:]
    t = _gelu(jnp.concatenate([y0, y1], axis=1))
    y = _ln(t, g2[...], bt2[...])
    z_ref[...] = _mm(y, Wm2[...]) + bm2[...] + zskip[...]


def kernel(x, edge_index, g1, bt1, Wm1, bm1, Wc1, bc1, Wc2, bc2,
           g2, bt2, Wm2, bm2, Ws, bs):
    src = edge_index[0].astype(jnp.int32)
    dst = edge_index[1].astype(jnp.int32)
    pad = EP - src.shape[0]
    pad_ids = jnp.arange(pad, dtype=jnp.int32)
    src2d = jnp.concatenate([src, pad_ids % N]).reshape(EP // CH, CH)
    dst2d = jnp.concatenate([dst, N + pad_ids % N_DUMMY]).reshape(EP // CH, CH)

    zeros1d = jnp.zeros((ROWS_PER_TILE,), _f32)
    ones128 = jnp.ones((CH,), _f32)
    zrows = jnp.zeros((ROWS_PER_TILE, DH), _f32)

    counts0, counts1 = _deg_call(dst2d, zeros1d, ones128)
    c0 = counts0.reshape(N_PAD, 1)
    c1 = counts1.reshape(N_PAD, 1)

    xp = jnp.pad(x, ((0, N_PAD - N), (0, 0)))

    g1r, bt1r = g1.reshape(1, D), bt1.reshape(1, D)
    g2r, bt2r = g2.reshape(1, D), bt2.reshape(1, D)
    bm1r, bc1r = bm1.reshape(1, D), bc1.reshape(1, D)
    bc2r, bm2r, bsr = bc2.reshape(1, D), bm2.reshape(1, D), bs.reshape(1, D)

    hv, zskip = pl.pallas_call(
        _tc_pre_a,
        grid=GRID,
        in_specs=[_rows(D), _full((1, D)), _full((1, D)), _full((D, D)),
                  _full((1, D)), _full((D, D)), _full((D, D)), _full((1, D))],
        out_specs=[_rows(D), _rows(D)],
        out_shape=[jax.ShapeDtypeStruct((N_PAD, D), _f32),
                   jax.ShapeDtypeStruct((N_PAD, D), _f32)],
    )(xp, g1r, bt1r, Wm1, bm1r, Wc1, Ws, bsr)

    xs0, xs1, dinv = pl.pallas_call(
        _tc_pre_b,
        grid=GRID,
        in_specs=[_rows(D), _rows(1), _rows(1)],
        out_specs=[_rows(DH), _rows(DH), _rows(1)],
        out_shape=[jax.ShapeDtypeStruct((N_PAD, DH), _f32),
                   jax.ShapeDtypeStruct((N_PAD, DH), _f32),
                   jax.ShapeDtypeStruct((N_PAD, 1), _f32)],
    )(hv, c0, c1)

    u0, u1 = _conv_call(xs0, xs1, src2d, dst2d, zrows)

    xs2_0, xs2_1 = pl.pallas_call(
        _tc_mid,
        grid=GRID,
        in_specs=[_rows(DH), _rows(DH), _rows(DH), _rows(DH), _rows(1),
                  _full((1, D)), _full((D, D))],
        out_specs=[_rows(DH), _rows(DH)],
        out_shape=[jax.ShapeDtypeStruct((N_PAD, DH), _f32),
                   jax.ShapeDtypeStruct((N_PAD, DH), _f32)],
    )(u0, u1, xs0, xs1, dinv, bc1r, Wc2)

    v0, v1 = _conv_call(xs2_0, xs2_1, src2d, dst2d, zrows)

    z = pl.pallas_call(
        _tc_post,
        grid=GRID,
        in_specs=[_rows(DH), _rows(DH), _rows(DH), _rows(DH), _rows(1),
                  _full((1, D)), _full((1, D)), _full((1, D)), _full((D, D)),
                  _full((1, D)), _rows(D)],
        out_specs=_rows(D),
        out_shape=jax.ShapeDtypeStruct((N_PAD, D), _f32),
    )(v0, v1, xs2_0, xs2_1, dinv, bc2r, g2r, bt2r, Wm2, bm2r, zskip)

    return z[:N]


# speedup vs baseline: 1.0319x; 1.0319x over previous
import functools

import jax
import jax.numpy as jnp
from jax import lax
from jax.experimental import pallas as pl
from jax.experimental.pallas import tpu as pltpu
from jax.experimental.pallas import tpu_sc as plsc

N = 10000
D = 256
DH = 128
N_PAD = 10240
EP = 163840
CH = 128
NCHUNK_CONV = 80
NCHUNK_HALF = 40
NCHUNK_DEG = 40
N_DUMMY = N_PAD - N
ROWS_PER_TILE = N_PAD // 16

_MESH = plsc.VectorSubcoreMesh(core_axis_name="c", subcore_axis_name="s")

_f32 = jnp.float32


def _deg_body(dst2d, zeros1d, ones_h, out0, out1, dst_idx, ones_v, counts):
    cid = lax.axis_index("c")
    sid = lax.axis_index("s")
    base = (cid * 16 + sid) * NCHUNK_DEG
    pltpu.sync_copy(dst2d.at[pl.ds(base, NCHUNK_DEG)], dst_idx)
    pltpu.sync_copy(ones_h, ones_v)
    pltpu.sync_copy(zeros1d, counts.at[pl.ds(sid * ROWS_PER_TILE, ROWS_PER_TILE)])
    plsc.subcore_barrier()

    def step(j, c):
        pltpu.sync_copy(ones_v, counts.at[dst_idx.at[j]], add=True)
        return c

    lax.fori_loop(0, NCHUNK_DEG, step, 0)
    plsc.subcore_barrier()

    @pl.when(cid == 0)
    def _():
        pltpu.sync_copy(counts.at[pl.ds(sid * ROWS_PER_TILE, ROWS_PER_TILE)],
                        out0.at[pl.ds(sid * ROWS_PER_TILE, ROWS_PER_TILE)])

    @pl.when(cid == 1)
    def _():
        pltpu.sync_copy(counts.at[pl.ds(sid * ROWS_PER_TILE, ROWS_PER_TILE)],
                        out1.at[pl.ds(sid * ROWS_PER_TILE, ROWS_PER_TILE)])


_deg_call = pl.kernel(
    _deg_body,
    out_type=(jax.ShapeDtypeStruct((N_PAD,), _f32),
              jax.ShapeDtypeStruct((N_PAD,), _f32)),
    mesh=_MESH,
    scratch_types=[
        pltpu.VMEM((NCHUNK_DEG, CH), jnp.int32),
        pltpu.VMEM((CH,), _f32),
        pltpu.VMEM_SHARED((N_PAD,), _f32),
    ],
)


def _conv_body(xs0, xs1, src2d, dst2d, zrows, out0, out1,
               src_idx, dst_idx, rb0, rb1, acc, sem0, sem1):
    cid = lax.axis_index("c")
    sid = lax.axis_index("s")
    row0 = sid * ROWS_PER_TILE
    pltpu.sync_copy(zrows, acc.at[pl.ds(row0, ROWS_PER_TILE)])
    plsc.subcore_barrier()

    def run(xs, out):
        for half in range(NCHUNK_CONV // NCHUNK_HALF):
            ibase = sid * NCHUNK_CONV + half * NCHUNK_HALF
            pltpu.sync_copy(src2d.at[pl.ds(ibase, NCHUNK_HALF)], src_idx)
            pltpu.sync_copy(dst2d.at[pl.ds(ibase, NCHUNK_HALF)], dst_idx)
            pltpu.async_copy(xs.at[src_idx.at[0]], rb0, sem0)

            def step(i, c):
                j0 = 2 * i
                pltpu.make_async_copy(xs.at[src_idx.at[j0]], rb0, sem0).wait()
                pltpu.async_copy(xs.at[src_idx.at[j0 + 1]], rb1, sem1)
                pltpu.sync_copy(rb0, acc.at[dst_idx.at[j0]], add=True)
                pltpu.make_async_copy(xs.at[src_idx.at[j0 + 1]], rb1, sem1).wait()

                @pl.when(i < NCHUNK_HALF // 2 - 1)
                def _():
                    pltpu.async_copy(xs.at[src_idx.at[j0 + 2]], rb0, sem0)

                pltpu.sync_copy(rb1, acc.at[dst_idx.at[j0 + 1]], add=True)
                return c

            lax.fori_loop(0, NCHUNK_HALF // 2, step, 0)
        plsc.subcore_barrier()
        pltpu.sync_copy(acc.at[pl.ds(row0, ROWS_PER_TILE)],
                        out.at[pl.ds(row0, ROWS_PER_TILE)])

    @pl.when(cid == 0)
    def _():
        run(xs0, out0)

    @pl.when(cid == 1)
    def _():
        run(xs1, out1)


_conv_call = pl.kernel(
    _conv_body,
    out_type=(jax.ShapeDtypeStruct((N_PAD, DH), _f32),
              jax.ShapeDtypeStruct((N_PAD, DH), _f32)),
    mesh=_MESH,
    scratch_types=[
        pltpu.VMEM((NCHUNK_HALF, CH), jnp.int32),
        pltpu.VMEM((NCHUNK_HALF, CH), jnp.int32),
        pltpu.VMEM((CH, DH), _f32),
        pltpu.VMEM((CH, DH), _f32),
        pltpu.VMEM_SHARED((N_PAD, DH), _f32),
        pltpu.SemaphoreType.DMA,
        pltpu.SemaphoreType.DMA,
    ],
)


BN = 2048
GRID = (N_PAD // BN,)
_HIGH = jax.lax.Precision.HIGHEST


def _rows(d):
    return pl.BlockSpec((BN, d), lambda i: (i, 0))


def _full(shape):
    return pl.BlockSpec(shape, lambda i: tuple(0 for _ in shape))


def _mm(a, b):
    return jnp.dot(a, b, preferred_element_type=_f32)


def _gelu(t):
    return t * 0.5 * (1.0 + lax.erf(t * 0.7071067811865476))


def _ln(t, g, b):
    m = jnp.mean(t, axis=1, keepdims=True)
    v = jnp.mean((t - m) ** 2, axis=1, keepdims=True)
    return (t - m) * lax.rsqrt(v + 1e-5) * g + b


def _tc_pre(x_ref, g1, bt1, Wm1, bm1, Wc1, Ws, bs, c0, c1,
            xs0, xs1, dinv_o, zskip):
    xb = x_ref[...]
    h = _ln(xb, g1[...], bt1[...])
    h = _mm(h, Wm1[...]) + bm1[...]
    hv = _mm(h, Wc1[...])
    deg = c0[...] + c1[...] + 1.0
    dinv = lax.rsqrt(deg)
    dinv_o[...] = dinv
    xs0[...] = hv[:, :DH] * dinv
    xs1[...] = hv[:, DH:] * dinv
    zskip[...] = _mm(xb, Ws[...]) + bs[...]


def _tc_mid(u0, u1, xs0, xs1, dinv, bc1, Wc2, xs2_0, xs2_1):
    dv = dinv[...]
    h0 = _gelu(dv * (u0[...] + xs0[...]) + bc1[...][:, :DH])
    h1 = _gelu(dv * (u1[...] + xs1[...]) + bc1[...][:, DH:])
    hv2 = _mm(h0, Wc2[...][:DH, :]) + _mm(h1, Wc2[...][DH:, :])
    xs2_0[...] = hv2[:, :DH] * dv
    xs2_1[...] = hv2[:, DH:] * dv


def _tc_post(u0, u1, xs0, xs1, dinv, bc2, g2, bt2, Wm2, bm2, zskip, z_ref):
    dv = dinv[...]
    y0 = dv * (u0[...] + xs0[...]) + bc2[...][:, :DH]
    y1 = dv * (u1[...] + xs1[...]) + bc2[...][:, DH:]
    t = _gelu(jnp.concatenate([y0, y1], axis=1))
    y = _ln(t, g2[...], bt2[...])
    z_ref[...] = _mm(y, Wm2[...]) + bm2[...] + zskip[...]


def kernel(x, edge_index, g1, bt1, Wm1, bm1, Wc1, bc1, Wc2, bc2,
           g2, bt2, Wm2, bm2, Ws, bs):
    src = edge_index[0].astype(jnp.int32)
    dst = edge_index[1].astype(jnp.int32)
    pad = EP - src.shape[0]
    pad_ids = jnp.arange(pad, dtype=jnp.int32)
    src2d = jnp.concatenate([src, pad_ids % N]).reshape(EP // CH, CH)
    dst2d = jnp.concatenate([dst, N + pad_ids % N_DUMMY]).reshape(EP // CH, CH)

    zeros1d = jnp.zeros((ROWS_PER_TILE,), _f32)
    ones128 = jnp.ones((CH,), _f32)
    zrows = jnp.zeros((ROWS_PER_TILE, DH), _f32)

    counts0, counts1 = _deg_call(dst2d, zeros1d, ones128)
    c0 = counts0.reshape(N_PAD, 1)
    c1 = counts1.reshape(N_PAD, 1)

    xp = jnp.pad(x, ((0, N_PAD - N), (0, 0)))

    g1r, bt1r = g1.reshape(1, D), bt1.reshape(1, D)
    g2r, bt2r = g2.reshape(1, D), bt2.reshape(1, D)
    bm1r, bc1r = bm1.reshape(1, D), bc1.reshape(1, D)
    bc2r, bm2r, bsr = bc2.reshape(1, D), bm2.reshape(1, D), bs.reshape(1, D)

    xs0, xs1, dinv, zskip = pl.pallas_call(
        _tc_pre,
        grid=GRID,
        in_specs=[_rows(D), _full((1, D)), _full((1, D)), _full((D, D)),
                  _full((1, D)), _full((D, D)), _full((D, D)), _full((1, D)),
                  _rows(1), _rows(1)],
        out_specs=[_rows(DH), _rows(DH), _rows(1), _rows(D)],
        out_shape=[jax.ShapeDtypeStruct((N_PAD, DH), _f32),
                   jax.ShapeDtypeStruct((N_PAD, DH), _f32),
                   jax.ShapeDtypeStruct((N_PAD, 1), _f32),
                   jax.ShapeDtypeStruct((N_PAD, D), _f32)],
    )(xp, g1r, bt1r, Wm1, bm1r, Wc1, Ws, bsr, c0, c1)

    u0, u1 = _conv_call(xs0, xs1, src2d, dst2d, zrows)

    xs2_0, xs2_1 = pl.pallas_call(
        _tc_mid,
        grid=GRID,
        in_specs=[_rows(DH), _rows(DH), _rows(DH), _rows(DH), _rows(1),
                  _full((1, D)), _full((D, D))],
        out_specs=[_rows(DH), _rows(DH)],
        out_shape=[jax.ShapeDtypeStruct((N_PAD, DH), _f32),
                   jax.ShapeDtypeStruct((N_PAD, DH), _f32)],
    )(u0, u1, xs0, xs1, dinv, bc1r, Wc2)

    v0, v1 = _conv_call(xs2_0, xs2_1, src2d, dst2d, zrows)

    z = pl.pallas_call(
        _tc_post,
        grid=GRID,
        in_specs=[_rows(DH), _rows(DH), _rows(DH), _rows(DH), _rows(1),
                  _full((1, D)), _full((1, D)), _full((1, D)), _full((D, D)),
                  _full((1, D)), _rows(D)],
        out_specs=_rows(D),
        out_shape=jax.ShapeDtypeStruct((N_PAD, D), _f32),
    )(v0, v1, xs2_0, xs2_1, dinv, bc2r, g2r, bt2r, Wm2, bm2r, zskip)

    return z[:N]


# speedup vs baseline: 1.0362x; 1.0042x over previous
import functools

import jax
import jax.numpy as jnp
from jax import lax
from jax.experimental import pallas as pl
from jax.experimental.pallas import tpu as pltpu
from jax.experimental.pallas import tpu_sc as plsc

N = 10000
D = 256
DH = 128
N_PAD = 10240
EP = 163840
CH = 128
NCHUNK_CONV = 80
NCHUNK_HALF = 40
NCHUNK_DEG = 40
N_DUMMY = N_PAD - N
ROWS_PER_TILE = N_PAD // 16

_MESH = plsc.VectorSubcoreMesh(core_axis_name="c", subcore_axis_name="s")

_f32 = jnp.float32


def _deg_body(dst2d, zeros1d, ones_h, out0, out1, dst_idx, ones_v, counts):
    cid = lax.axis_index("c")
    sid = lax.axis_index("s")
    base = (cid * 16 + sid) * NCHUNK_DEG
    pltpu.sync_copy(dst2d.at[pl.ds(base, NCHUNK_DEG)], dst_idx)
    pltpu.sync_copy(ones_h, ones_v)
    pltpu.sync_copy(zeros1d, counts.at[pl.ds(sid * ROWS_PER_TILE, ROWS_PER_TILE)])
    plsc.subcore_barrier()

    def step(j, c):
        pltpu.sync_copy(ones_v, counts.at[dst_idx.at[j]], add=True)
        return c

    lax.fori_loop(0, NCHUNK_DEG, step, 0)
    plsc.subcore_barrier()

    @pl.when(cid == 0)
    def _():
        pltpu.sync_copy(counts.at[pl.ds(sid * ROWS_PER_TILE, ROWS_PER_TILE)],
                        out0.at[pl.ds(sid * ROWS_PER_TILE, ROWS_PER_TILE)])

    @pl.when(cid == 1)
    def _():
        pltpu.sync_copy(counts.at[pl.ds(sid * ROWS_PER_TILE, ROWS_PER_TILE)],
                        out1.at[pl.ds(sid * ROWS_PER_TILE, ROWS_PER_TILE)])


_deg_call = pl.kernel(
    _deg_body,
    out_type=(jax.ShapeDtypeStruct((N_PAD,), _f32),
              jax.ShapeDtypeStruct((N_PAD,), _f32)),
    mesh=_MESH,
    scratch_types=[
        pltpu.VMEM((NCHUNK_DEG, CH), jnp.int32),
        pltpu.VMEM((CH,), _f32),
        pltpu.VMEM_SHARED((N_PAD,), _f32),
    ],
)


def _conv_body(xs0, xs1, src2d, dst2d, zrows, out0, out1,
               src_idx, dst_idx, rb0, rb1, acc, sem0, sem1):
    cid = lax.axis_index("c")
    sid = lax.axis_index("s")
    row0 = sid * ROWS_PER_TILE
    pltpu.sync_copy(zrows, acc.at[pl.ds(row0, ROWS_PER_TILE)])
    plsc.subcore_barrier()

    def run(xs, out):
        for half in range(NCHUNK_CONV // NCHUNK_HALF):
            ibase = sid * NCHUNK_CONV + half * NCHUNK_HALF
            pltpu.sync_copy(src2d.at[pl.ds(ibase, NCHUNK_HALF)], src_idx)
            pltpu.sync_copy(dst2d.at[pl.ds(ibase, NCHUNK_HALF)], dst_idx)
            pltpu.async_copy(xs.at[src_idx.at[0]], rb0, sem0)

            def step(i, c):
                j0 = 2 * i
                pltpu.make_async_copy(xs.at[src_idx.at[j0]], rb0, sem0).wait()
                pltpu.async_copy(xs.at[src_idx.at[j0 + 1]], rb1, sem1)
                pltpu.sync_copy(rb0, acc.at[dst_idx.at[j0]], add=True)
                pltpu.make_async_copy(xs.at[src_idx.at[j0 + 1]], rb1, sem1).wait()

                @pl.when(i < NCHUNK_HALF // 2 - 1)
                def _():
                    pltpu.async_copy(xs.at[src_idx.at[j0 + 2]], rb0, sem0)

                pltpu.sync_copy(rb1, acc.at[dst_idx.at[j0 + 1]], add=True)
                return c

            lax.fori_loop(0, NCHUNK_HALF // 2, step, 0)
        plsc.subcore_barrier()
        pltpu.sync_copy(acc.at[pl.ds(row0, ROWS_PER_TILE)],
                        out.at[pl.ds(row0, ROWS_PER_TILE)])

    @pl.when(cid == 0)
    def _():
        run(xs0, out0)

    @pl.when(cid == 1)
    def _():
        run(xs1, out1)


_conv_call = pl.kernel(
    _conv_body,
    out_type=(jax.ShapeDtypeStruct((N_PAD, DH), _f32),
              jax.ShapeDtypeStruct((N_PAD, DH), _f32)),
    mesh=_MESH,
    scratch_types=[
        pltpu.VMEM((NCHUNK_HALF, CH), jnp.int32),
        pltpu.VMEM((NCHUNK_HALF, CH), jnp.int32),
        pltpu.VMEM((CH, DH), _f32),
        pltpu.VMEM((CH, DH), _f32),
        pltpu.VMEM_SHARED((N_PAD, DH), _f32),
        pltpu.SemaphoreType.DMA,
        pltpu.SemaphoreType.DMA,
    ],
)


BN = 5120
GRID = (N_PAD // BN,)
_HIGH = jax.lax.Precision.HIGHEST


def _rows(d):
    return pl.BlockSpec((BN, d), lambda i: (i, 0))


def _full(shape):
    return pl.BlockSpec(shape, lambda i: tuple(0 for _ in shape))


def _mm(a, b):
    return jnp.dot(a, b, preferred_element_type=_f32)


def _gelu(t):
    return t * 0.5 * (1.0 + lax.erf(t * 0.7071067811865476))


def _ln(t, g, b):
    m = jnp.mean(t, axis=1, keepdims=True)
    v = jnp.mean((t - m) ** 2, axis=1, keepdims=True)
    return (t - m) * lax.rsqrt(v + 1e-5) * g + b


def _tc_pre(x_ref, g1, bt1, Wm1, bm1, Wc1, Ws, bs, c0, c1,
            xs0, xs1, dinv_o, zskip):
    xb = x_ref[...]
    h = _ln(xb, g1[...], bt1[...])
    h = _mm(h, Wm1[...]) + bm1[...]
    hv = _mm(h, Wc1[...])
    deg = c0[...] + c1[...] + 1.0
    dinv = lax.rsqrt(deg)
    dinv_o[...] = dinv
    xs0[...] = hv[:, :DH] * dinv
    xs1[...] = hv[:, DH:] * dinv
    zskip[...] = _mm(xb, Ws[...]) + bs[...]


def _tc_mid(u0, u1, xs0, xs1, dinv, bc1, Wc2, xs2_0, xs2_1):
    dv = dinv[...]
    h0 = _gelu(dv * (u0[...] + xs0[...]) + bc1[...][:, :DH])
    h1 = _gelu(dv * (u1[...] + xs1[...]) + bc1[...][:, DH:])
    hv2 = _mm(h0, Wc2[...][:DH, :]) + _mm(h1, Wc2[...][DH:, :])
    xs2_0[...] = hv2[:, :DH] * dv
    xs2_1[...] = hv2[:, DH:] * dv


def _tc_post(u0, u1, xs0, xs1, dinv, bc2, g2, bt2, Wm2, bm2, zskip, z_ref):
    dv = dinv[...]
    y0 = dv * (u0[...] + xs0[...]) + bc2[...][:, :DH]
    y1 = dv * (u1[...] + xs1[...]) + bc2[...][:, DH:]
    t = _gelu(jnp.concatenate([y0, y1], axis=1))
    y = _ln(t, g2[...], bt2[...])
    z_ref[...] = _mm(y, Wm2[...]) + bm2[...] + zskip[...]


def kernel(x, edge_index, g1, bt1, Wm1, bm1, Wc1, bc1, Wc2, bc2,
           g2, bt2, Wm2, bm2, Ws, bs):
    src = edge_index[0].astype(jnp.int32)
    dst = edge_index[1].astype(jnp.int32)
    pad = EP - src.shape[0]
    pad_ids = jnp.arange(pad, dtype=jnp.int32)
    src2d = jnp.concatenate([src, pad_ids % N]).reshape(EP // CH, CH)
    dst2d = jnp.concatenate([dst, N + pad_ids % N_DUMMY]).reshape(EP // CH, CH)

    zeros1d = jnp.zeros((ROWS_PER_TILE,), _f32)
    ones128 = jnp.ones((CH,), _f32)
    zrows = jnp.zeros((ROWS_PER_TILE, DH), _f32)

    counts0, counts1 = _deg_call(dst2d, zeros1d, ones128)
    c0 = counts0.reshape(N_PAD, 1)
    c1 = counts1.reshape(N_PAD, 1)

    xp = jnp.pad(x, ((0, N_PAD - N), (0, 0)))

    g1r, bt1r = g1.reshape(1, D), bt1.reshape(1, D)
    g2r, bt2r = g2.reshape(1, D), bt2.reshape(1, D)
    bm1r, bc1r = bm1.reshape(1, D), bc1.reshape(1, D)
    bc2r, bm2r, bsr = bc2.reshape(1, D), bm2.reshape(1, D), bs.reshape(1, D)

    xs0, xs1, dinv, zskip = pl.pallas_call(
        _tc_pre,
        grid=GRID,
        in_specs=[_rows(D), _full((1, D)), _full((1, D)), _full((D, D)),
                  _full((1, D)), _full((D, D)), _full((D, D)), _full((1, D)),
                  _rows(1), _rows(1)],
        out_specs=[_rows(DH), _rows(DH), _rows(1), _rows(D)],
        out_shape=[jax.ShapeDtypeStruct((N_PAD, DH), _f32),
                   jax.ShapeDtypeStruct((N_PAD, DH), _f32),
                   jax.ShapeDtypeStruct((N_PAD, 1), _f32),
                   jax.ShapeDtypeStruct((N_PAD, D), _f32)],
    )(xp, g1r, bt1r, Wm1, bm1r, Wc1, Ws, bsr, c0, c1)

    u0, u1 = _conv_call(xs0, xs1, src2d, dst2d, zrows)

    xs2_0, xs2_1 = pl.pallas_call(
        _tc_mid,
        grid=GRID,
        in_specs=[_rows(DH), _rows(DH), _rows(DH), _rows(DH), _rows(1),
                  _full((1, D)), _full((D, D))],
        out_specs=[_rows(DH), _rows(DH)],
        out_shape=[jax.ShapeDtypeStruct((N_PAD, DH), _f32),
                   jax.ShapeDtypeStruct((N_PAD, DH), _f32)],
    )(u0, u1, xs0, xs1, dinv, bc1r, Wc2)

    v0, v1 = _conv_call(xs2_0, xs2_1, src2d, dst2d, zrows)

    z = pl.pallas_call(
        _tc_post,
        grid=GRID,
        in_specs=[_rows(DH), _rows(DH), _rows(DH), _rows(DH), _rows(1),
                  _full((1, D)), _full((1, D)), _full((1, D)), _full((D, D)),
                  _full((1, D)), _rows(D)],
        out_specs=_rows(D),
        out_shape=jax.ShapeDtypeStruct((N_PAD, D), _f32),
    )(v0, v1, xs2_0, xs2_1, dinv, bc2r, g2r, bt2r, Wm2, bm2r, zskip)

    return z[:N]
